# baseline (speedup 1.0000x reference)
import functools

import jax
import jax.numpy as jnp
from jax import lax
from jax.experimental import pallas as pl
from jax.experimental.pallas import tpu as pltpu
from jax.experimental.pallas import tpu_sc as plsc

_VOCAB = 100000
_N_EMB = _VOCAB + 2
_DIM = 128
_B = 4096
_L = 200
_PADDING_ID = _N_EMB - 1

_CH = 50
_NCH = _L // _CH
_NV = _DIM // 16

_NWORKERS = 32
_ROWS_PER_W = _B // _NWORKERS


def _pooler_body(ids_hbm, pads_hbm, table_hbm, out_hbm,
                 ids_v, pads_v, buf0_v, buf1_v, outrow_v,
                 sem0, sem1, semw0, semw1):
    nc = 2
    wid = lax.axis_index("s") * nc + lax.axis_index("c")
    base = wid * _ROWS_PER_W

    pltpu.sync_copy(ids_hbm.at[pl.ds(base * _NCH, _ROWS_PER_W * _NCH)], ids_v)
    pltpu.sync_copy(pads_hbm.at[pl.ds(base, _ROWS_PER_W)], pads_v)
    pltpu.sync_copy(table_hbm.at[pl.ds(_PADDING_ID, 1)],
                    outrow_v.at[pl.ds(0, 1), pl.ds(0, _DIM)])
    pv = tuple(outrow_v[0, pl.ds(16 * v, 16)] for v in range(_NV))

    neg_inf = jnp.full((16,), -jnp.inf, jnp.float32)
    zero = jnp.zeros((16,), jnp.float32)
    lane = lax.iota(jnp.int32, 16)

    def seq_len(r):
        g = lax.shift_right_logical(r, 4)
        l = r - g * 16
        pvec = pads_v[pl.ds(g * 16, 16)]
        pads_r = jnp.max(jnp.where(lane == l, pvec, 0))
        return _L - pads_r

    def gather_plan(n):
        nfull = jnp.int32(0)
        for k in range(1, _NCH + 1):
            nfull = nfull + (n >= k * _CH).astype(jnp.int32)
        m = n - nfull * _CH
        nfull = nfull + (m >= 49).astype(jnp.int32)
        m = jnp.maximum(n - nfull * _CH, 0)
        tail_sel = lax.shift_right_logical(m + 7, 3)
        return nfull, tail_sel

    def each_gather(r, buf, n, go):
        nfull, tail_sel = gather_plan(n)
        for c in range(_NCH):
            @pl.when(c < nfull)
            def _():
                go(table_hbm.at[ids_v.at[r * _NCH + c]],
                   buf.at[pl.ds(c * _CH, _CH)])
        for t in range(1, 7):
            @pl.when(tail_sel == t)
            def _():
                go(table_hbm.at[ids_v.at[r * _NCH + nfull, pl.ds(0, 8 * t)]],
                   buf.at[pl.ds(nfull * _CH, 8 * t)])

    def fire_row(r, buf, sem, n):
        each_gather(r, buf, n,
                    lambda src, dst: pltpu.async_copy(src, dst, sem))

    def compute_row(r, buf, sem, p, semw, pv, n):
        each_gather(r, buf, n,
                    lambda src, dst: pltpu.make_async_copy(src, dst,
                                                           sem).wait())

        pads_r = _L - n
        acc = ((zero,) * _NV,
               tuple(jnp.where(pads_r > 0, pv[v], neg_inf)
                     for v in range(_NV)))

        def body(j, acc2):
            sums, maxs = acc2
            ra = tuple(buf[j, pl.ds(16 * v, 16)] for v in range(_NV))
            rb = tuple(buf[j + 1, pl.ds(16 * v, 16)] for v in range(_NV))
            return (tuple(sums[v] + (ra[v] + rb[v]) for v in range(_NV)),
                    tuple(jnp.maximum(maxs[v], jnp.maximum(ra[v], rb[v]))
                          for v in range(_NV)))

        odd = (n & 1) == 1
        n_even = n - (n & 1)
        sums, maxs = plsc.parallel_loop(0, n_even, 2, unroll=2,
                                        carry=acc)(body)
        rt = tuple(buf[n - 1, pl.ds(16 * v, 16)] for v in range(_NV))
        sums = tuple(sums[v] + jnp.where(odd, rt[v], zero)
                     for v in range(_NV))
        maxs = tuple(jnp.maximum(maxs[v], jnp.where(odd, rt[v], neg_inf))
                     for v in range(_NV))
        n_vec = jnp.full((16,), 1.0, jnp.float32) * n.astype(jnp.float32)
        inv = jnp.full((16,), 1.0, jnp.float32) / n_vec
        @pl.when(r >= 2)
        def _():
            pltpu.make_async_copy(outrow_v.at[p], out_hbm.at[base + r - 2],
                                  semw).wait()
        for v in range(_NV):
            outrow_v[p, pl.ds(16 * v, 16)] = maxs[v]
            outrow_v[p, pl.ds(_DIM + 16 * v, 16)] = sums[v] * inv
        pltpu.async_copy(outrow_v.at[p], out_hbm.at[base + r], semw)

    n0 = seq_len(jnp.int32(0))
    fire_row(jnp.int32(0), buf0_v, sem0, n0)

    def pair_body(rp, carry):
        pv_c, n_r0 = carry
        r0 = rp * 2
        n_r1 = seq_len(r0 + 1)
        fire_row(r0 + 1, buf1_v, sem1, n_r1)
        compute_row(r0, buf0_v, sem0, 0, semw0, pv_c, n_r0)

        n_r2 = seq_len(jnp.minimum(r0 + 2, _ROWS_PER_W - 1))
        @pl.when(r0 + 2 < _ROWS_PER_W)
        def _():
            fire_row(r0 + 2, buf0_v, sem0, n_r2)
        compute_row(r0 + 1, buf1_v, sem1, 1, semw1, pv_c, n_r1)
        return (pv_c, n_r2)

    lax.fori_loop(0, _ROWS_PER_W // 2, pair_body, (pv, n0))
    pltpu.make_async_copy(outrow_v.at[0],
                          out_hbm.at[base + _ROWS_PER_W - 2], semw0).wait()
    pltpu.make_async_copy(outrow_v.at[1],
                          out_hbm.at[base + _ROWS_PER_W - 1], semw1).wait()


@jax.jit
def _pooler(ids2, pads, emb_table):
    mesh = plsc.VectorSubcoreMesh(core_axis_name="c", subcore_axis_name="s")
    run = pl.kernel(
        _pooler_body,
        out_type=jax.ShapeDtypeStruct((_B, 2 * _DIM), jnp.float32),
        mesh=mesh,
        scratch_types=[
            pltpu.VMEM((_ROWS_PER_W * _NCH, _CH), jnp.int32),
            pltpu.VMEM((_ROWS_PER_W,), jnp.int32),
            pltpu.VMEM((_L, _DIM), jnp.float32),
            pltpu.VMEM((_L, _DIM), jnp.float32),
            pltpu.VMEM((2, 2 * _DIM), jnp.float32),
            pltpu.SemaphoreType.DMA,
            pltpu.SemaphoreType.DMA,
            pltpu.SemaphoreType.DMA,
            pltpu.SemaphoreType.DMA,
        ],
        compiler_params=pltpu.CompilerParams(needs_layout_passes=False),
    )
    return run(ids2, pads, emb_table)


def kernel(ids, pads, emb_table):
    ids2 = ids.reshape(_B * _NCH, _CH)
    return _pooler(ids2, pads, emb_table)


# speedup vs baseline: 1.1601x; 1.1601x over previous
import functools

import jax
import jax.numpy as jnp
from jax import lax
from jax.experimental import pallas as pl
from jax.experimental.pallas import tpu as pltpu
from jax.experimental.pallas import tpu_sc as plsc

_VOCAB = 100000
_N_EMB = _VOCAB + 2
_DIM = 128
_B = 4096
_L = 200
_PADDING_ID = _N_EMB - 1

_CH = 40
_NCH = _L // _CH
_NV = _DIM // 16

_NWORKERS = 32
_ROWS_PER_W = _B // _NWORKERS


def _pooler_body(ids_hbm, pads_hbm, table_hbm, out_hbm,
                 ids_v, pads_v, buf0_v, buf1_v, buf2_v, outrow_v,
                 sem0, sem1, sem2, semw0, semw1):
    nc = 2
    wid = lax.axis_index("s") * nc + lax.axis_index("c")
    base = wid * _ROWS_PER_W

    pltpu.sync_copy(ids_hbm.at[pl.ds(base * _L, _ROWS_PER_W * _L)], ids_v)
    pltpu.sync_copy(pads_hbm.at[pl.ds(base, _ROWS_PER_W)], pads_v)
    pltpu.sync_copy(table_hbm.at[pl.ds(_PADDING_ID, 1)],
                    outrow_v.at[pl.ds(0, 1), pl.ds(0, _DIM)])
    pv = tuple(outrow_v[0, pl.ds(16 * v, 16)] for v in range(_NV))

    neg_inf = jnp.full((16,), -jnp.inf, jnp.float32)
    zero = jnp.zeros((16,), jnp.float32)
    lane = lax.iota(jnp.int32, 16)

    def seq_len(r):
        g = lax.shift_right_logical(r, 4)
        l = r - g * 16
        pvec = pads_v[pl.ds(g * 16, 16)]
        pads_r = jnp.max(jnp.where(lane == l, pvec, 0))
        return _L - pads_r

    def gather_plan(n):
        nfull = jnp.int32(0)
        for k in range(1, _NCH + 1):
            nfull = nfull + (n >= k * _CH).astype(jnp.int32)
        m = n - nfull * _CH
        tail_sel = lax.shift_right_logical(m + 7, 3)
        return nfull, tail_sel

    def each_gather(r, buf, n, go):
        nfull, tail_sel = gather_plan(n)
        for c in range(_NCH):
            @pl.when(c < nfull)
            def _():
                go(table_hbm.at[ids_v.at[pl.ds(r * _L + c * _CH, _CH)]],
                   buf.at[pl.ds(c * _CH, _CH)])
        for t in range(1, _CH // 8 + 1):
            @pl.when(tail_sel == t)
            def _():
                go(table_hbm.at[ids_v.at[pl.ds(r * _L + nfull * _CH, 8 * t)]],
                   buf.at[pl.ds(nfull * _CH, 8 * t)])

    def fire_row(r, buf, sem, n):
        each_gather(r, buf, n,
                    lambda src, dst: pltpu.async_copy(src, dst, sem))

    def compute_row(r, buf, sem, p, semw, pv, n):
        each_gather(r, buf, n,
                    lambda src, dst: pltpu.make_async_copy(src, dst,
                                                           sem).wait())

        pads_r = _L - n
        acc = ((zero,) * _NV,
               tuple(jnp.where(pads_r > 0, pv[v], neg_inf)
                     for v in range(_NV)))

        def body(j, acc2):
            sums, maxs = acc2
            ra = tuple(buf[j, pl.ds(16 * v, 16)] for v in range(_NV))
            rb = tuple(buf[j + 1, pl.ds(16 * v, 16)] for v in range(_NV))
            return (tuple(sums[v] + (ra[v] + rb[v]) for v in range(_NV)),
                    tuple(jnp.maximum(maxs[v], jnp.maximum(ra[v], rb[v]))
                          for v in range(_NV)))

        odd = (n & 1) == 1
        n_even = n - (n & 1)
        sums, maxs = plsc.parallel_loop(0, n_even, 2, unroll=2,
                                        carry=acc)(body)
        rt = tuple(buf[n - 1, pl.ds(16 * v, 16)] for v in range(_NV))
        sums = tuple(sums[v] + jnp.where(odd, rt[v], zero)
                     for v in range(_NV))
        maxs = tuple(jnp.maximum(maxs[v], jnp.where(odd, rt[v], neg_inf))
                     for v in range(_NV))
        n_vec = jnp.full((16,), 1.0, jnp.float32) * n.astype(jnp.float32)
        inv = jnp.full((16,), 1.0, jnp.float32) / n_vec
        @pl.when(r >= 2)
        def _():
            pltpu.make_async_copy(outrow_v.at[p], out_hbm.at[base + r - 2],
                                  semw).wait()
        for v in range(_NV):
            outrow_v[p, pl.ds(16 * v, 16)] = maxs[v]
            outrow_v[p, pl.ds(_DIM + 16 * v, 16)] = sums[v] * inv
        pltpu.async_copy(outrow_v.at[p], out_hbm.at[base + r], semw)

    bufs = (buf0_v, buf1_v, buf2_v)
    sems = (sem0, sem1, sem2)

    def fire_guarded(r, buf, sem):
        @pl.when(r < _ROWS_PER_W)
        def _():
            fire_row(r, buf, sem, seq_len(jnp.minimum(r, _ROWS_PER_W - 1)))

    fire_row(jnp.int32(0), buf0_v, sem0, seq_len(jnp.int32(0)))
    fire_row(jnp.int32(1), buf1_v, sem1, seq_len(jnp.int32(1)))

    def triple_body(tp, pv_c):
        r0 = tp * 3
        for k in range(3):
            fire_guarded(r0 + k + 2, bufs[(k + 2) % 3], sems[(k + 2) % 3])
            compute_row(r0 + k, bufs[k], sems[k], k & 1, (semw0, semw1)[k & 1],
                        pv_c, seq_len(r0 + k))
        return pv_c

    lax.fori_loop(0, _ROWS_PER_W // 3, triple_body, pv)
    r126 = jnp.int32(_ROWS_PER_W - 2)
    r127 = jnp.int32(_ROWS_PER_W - 1)
    compute_row(r126, buf0_v, sem0, 0, semw0, pv, seq_len(r126))
    compute_row(r127, buf1_v, sem1, 1, semw1, pv, seq_len(r127))
    pltpu.make_async_copy(outrow_v.at[0],
                          out_hbm.at[base + _ROWS_PER_W - 2], semw0).wait()
    pltpu.make_async_copy(outrow_v.at[1],
                          out_hbm.at[base + _ROWS_PER_W - 1], semw1).wait()


@jax.jit
def _pooler(ids2, pads, emb_table):
    mesh = plsc.VectorSubcoreMesh(core_axis_name="c", subcore_axis_name="s")
    run = pl.kernel(
        _pooler_body,
        out_type=jax.ShapeDtypeStruct((_B, 2 * _DIM), jnp.float32),
        mesh=mesh,
        scratch_types=[
            pltpu.VMEM((_ROWS_PER_W * _L,), jnp.int32),
            pltpu.VMEM((_ROWS_PER_W,), jnp.int32),
            pltpu.VMEM((_L, _DIM), jnp.float32),
            pltpu.VMEM((_L, _DIM), jnp.float32),
            pltpu.VMEM((_L, _DIM), jnp.float32),
            pltpu.VMEM((2, 2 * _DIM), jnp.float32),
            pltpu.SemaphoreType.DMA,
            pltpu.SemaphoreType.DMA,
            pltpu.SemaphoreType.DMA,
            pltpu.SemaphoreType.DMA,
            pltpu.SemaphoreType.DMA,
        ],
        compiler_params=pltpu.CompilerParams(needs_layout_passes=False),
    )
    return run(ids2, pads, emb_table)


def kernel(ids, pads, emb_table):
    ids2 = ids.reshape(_B * _L)
    return _pooler(ids2, pads, emb_table)


# speedup vs baseline: 1.1657x; 1.0048x over previous
import functools

import jax
import jax.numpy as jnp
from jax import lax
from jax.experimental import pallas as pl
from jax.experimental.pallas import tpu as pltpu
from jax.experimental.pallas import tpu_sc as plsc

_VOCAB = 100000
_N_EMB = _VOCAB + 2
_DIM = 128
_B = 4096
_L = 200
_PADDING_ID = _N_EMB - 1

_CH = 40
_NCH = _L // _CH
_NV = _DIM // 16

_NWORKERS = 32
_ROWS_PER_W = _B // _NWORKERS


def _pooler_body(ids_hbm, pads_hbm, table_hbm, out_hbm,
                 ids_v, pads_v, buf0_v, buf1_v, buf2_v, outrow_v,
                 sem0, sem1, sem2, semw0, semw1):
    nc = 2
    wid = lax.axis_index("s") * nc + lax.axis_index("c")
    base = wid * _ROWS_PER_W

    pltpu.sync_copy(ids_hbm.at[pl.ds(base * _L, _ROWS_PER_W * _L)], ids_v)
    pltpu.sync_copy(pads_hbm.at[pl.ds(base, _ROWS_PER_W)], pads_v)
    pltpu.sync_copy(table_hbm.at[pl.ds(_PADDING_ID, 1)],
                    outrow_v.at[pl.ds(0, 1), pl.ds(0, _DIM)])
    pv = tuple(outrow_v[0, pl.ds(16 * v, 16)] for v in range(_NV))

    neg_inf = jnp.full((16,), -jnp.inf, jnp.float32)
    zero = jnp.zeros((16,), jnp.float32)
    lane = lax.iota(jnp.int32, 16)

    def seq_len(r):
        g = lax.shift_right_logical(r, 4)
        l = r - g * 16
        pvec = pads_v[pl.ds(g * 16, 16)]
        pads_r = jnp.max(jnp.where(lane == l, pvec, 0))
        return _L - pads_r

    def gather_plan(n):
        nfull = jnp.int32(0)
        for k in range(1, _NCH + 1):
            nfull = nfull + (n >= k * _CH).astype(jnp.int32)
        m = n - nfull * _CH
        tail8 = lax.shift_right_logical(m, 3)
        rem = m & 7
        return nfull, tail8, rem

    def each_gather(r, buf, n, go):
        nfull, tail8, rem = gather_plan(n)
        for c in range(_NCH):
            @pl.when(c < nfull)
            def _():
                go(table_hbm.at[ids_v.at[pl.ds(r * _L + c * _CH, _CH)]],
                   buf.at[pl.ds(c * _CH, _CH)])
        for t in range(1, _CH // 8):
            @pl.when(tail8 == t)
            def _():
                go(table_hbm.at[ids_v.at[pl.ds(r * _L + nfull * _CH, 8 * t)]],
                   buf.at[pl.ds(nfull * _CH, 8 * t)])
        off = nfull * _CH + tail8 * 8
        for q in range(1, 8):
            @pl.when(rem == q)
            def _():
                go(table_hbm.at[ids_v.at[pl.ds(r * _L + off, q)]],
                   buf.at[pl.ds(off, q)])

    def fire_row(r, buf, sem, n):
        each_gather(r, buf, n,
                    lambda src, dst: pltpu.async_copy(src, dst, sem))

    def compute_row(r, buf, sem, p, semw, pv, n):
        each_gather(r, buf, n,
                    lambda src, dst: pltpu.make_async_copy(src, dst,
                                                           sem).wait())

        pads_r = _L - n
        acc = ((zero,) * _NV,
               tuple(jnp.where(pads_r > 0, pv[v], neg_inf)
                     for v in range(_NV)))

        def body(j, acc2):
            sums, maxs = acc2
            ra = tuple(buf[j, pl.ds(16 * v, 16)] for v in range(_NV))
            rb = tuple(buf[j + 1, pl.ds(16 * v, 16)] for v in range(_NV))
            return (tuple(sums[v] + (ra[v] + rb[v]) for v in range(_NV)),
                    tuple(jnp.maximum(maxs[v], jnp.maximum(ra[v], rb[v]))
                          for v in range(_NV)))

        odd = (n & 1) == 1
        n_even = n - (n & 1)
        sums, maxs = plsc.parallel_loop(0, n_even, 2, unroll=2,
                                        carry=acc)(body)
        rt = tuple(buf[n - 1, pl.ds(16 * v, 16)] for v in range(_NV))
        sums = tuple(sums[v] + jnp.where(odd, rt[v], zero)
                     for v in range(_NV))
        maxs = tuple(jnp.maximum(maxs[v], jnp.where(odd, rt[v], neg_inf))
                     for v in range(_NV))
        n_vec = jnp.full((16,), 1.0, jnp.float32) * n.astype(jnp.float32)
        inv = jnp.full((16,), 1.0, jnp.float32) / n_vec
        @pl.when(r >= 2)
        def _():
            pltpu.make_async_copy(outrow_v.at[p], out_hbm.at[base + r - 2],
                                  semw).wait()
        for v in range(_NV):
            outrow_v[p, pl.ds(16 * v, 16)] = maxs[v]
            outrow_v[p, pl.ds(_DIM + 16 * v, 16)] = sums[v] * inv
        pltpu.async_copy(outrow_v.at[p], out_hbm.at[base + r], semw)

    bufs = (buf0_v, buf1_v, buf2_v)
    sems = (sem0, sem1, sem2)

    def fire_guarded(r, buf, sem):
        @pl.when(r < _ROWS_PER_W)
        def _():
            fire_row(r, buf, sem, seq_len(jnp.minimum(r, _ROWS_PER_W - 1)))

    fire_row(jnp.int32(0), buf0_v, sem0, seq_len(jnp.int32(0)))
    fire_row(jnp.int32(1), buf1_v, sem1, seq_len(jnp.int32(1)))

    def triple_body(tp, pv_c):
        r0 = tp * 3
        for k in range(3):
            fire_guarded(r0 + k + 2, bufs[(k + 2) % 3], sems[(k + 2) % 3])
            compute_row(r0 + k, bufs[k], sems[k], k & 1, (semw0, semw1)[k & 1],
                        pv_c, seq_len(r0 + k))
        return pv_c

    lax.fori_loop(0, _ROWS_PER_W // 3, triple_body, pv)
    r126 = jnp.int32(_ROWS_PER_W - 2)
    r127 = jnp.int32(_ROWS_PER_W - 1)
    compute_row(r126, buf0_v, sem0, 0, semw0, pv, seq_len(r126))
    compute_row(r127, buf1_v, sem1, 1, semw1, pv, seq_len(r127))
    pltpu.make_async_copy(outrow_v.at[0],
                          out_hbm.at[base + _ROWS_PER_W - 2], semw0).wait()
    pltpu.make_async_copy(outrow_v.at[1],
                          out_hbm.at[base + _ROWS_PER_W - 1], semw1).wait()


@jax.jit
def _pooler(ids2, pads, emb_table):
    mesh = plsc.VectorSubcoreMesh(core_axis_name="c", subcore_axis_name="s")
    run = pl.kernel(
        _pooler_body,
        out_type=jax.ShapeDtypeStruct((_B, 2 * _DIM), jnp.float32),
        mesh=mesh,
        scratch_types=[
            pltpu.VMEM((_ROWS_PER_W * _L,), jnp.int32),
            pltpu.VMEM((_ROWS_PER_W,), jnp.int32),
            pltpu.VMEM((_L, _DIM), jnp.float32),
            pltpu.VMEM((_L, _DIM), jnp.float32),
            pltpu.VMEM((_L, _DIM), jnp.float32),
            pltpu.VMEM((2, 2 * _DIM), jnp.float32),
            pltpu.SemaphoreType.DMA,
            pltpu.SemaphoreType.DMA,
            pltpu.SemaphoreType.DMA,
            pltpu.SemaphoreType.DMA,
            pltpu.SemaphoreType.DMA,
        ],
        compiler_params=pltpu.CompilerParams(needs_layout_passes=False),
    )
    return run(ids2, pads, emb_table)


def kernel(ids, pads, emb_table):
    ids2 = ids.reshape(_B * _L)
    return _pooler(ids2, pads, emb_table)


# speedup vs baseline: 1.1675x; 1.0015x over previous
import functools

import jax
import jax.numpy as jnp
from jax import lax
from jax.experimental import pallas as pl
from jax.experimental.pallas import tpu as pltpu
from jax.experimental.pallas import tpu_sc as plsc

_VOCAB = 100000
_N_EMB = _VOCAB + 2
_DIM = 128
_B = 4096
_L = 200
_PADDING_ID = _N_EMB - 1

_CH = 48
_NCH = _L // _CH
_NV = _DIM // 16

_NWORKERS = 32
_ROWS_PER_W = _B // _NWORKERS


def _pooler_body(ids_hbm, pads_hbm, table_hbm, out_hbm,
                 ids_v, pads_v, buf0_v, buf1_v, buf2_v, outrow_v,
                 sem0, sem1, sem2, semw0, semw1):
    nc = 2
    wid = lax.axis_index("s") * nc + lax.axis_index("c")
    base = wid * _ROWS_PER_W

    pltpu.sync_copy(ids_hbm.at[pl.ds(base * _L, _ROWS_PER_W * _L)], ids_v)
    pltpu.sync_copy(pads_hbm.at[pl.ds(base, _ROWS_PER_W)], pads_v)
    pltpu.sync_copy(table_hbm.at[pl.ds(_PADDING_ID, 1)],
                    outrow_v.at[pl.ds(0, 1), pl.ds(0, _DIM)])
    pv = tuple(outrow_v[0, pl.ds(16 * v, 16)] for v in range(_NV))

    neg_inf = jnp.full((16,), -jnp.inf, jnp.float32)
    zero = jnp.zeros((16,), jnp.float32)
    lane = lax.iota(jnp.int32, 16)

    def seq_len(r):
        g = lax.shift_right_logical(r, 4)
        l = r - g * 16
        pvec = pads_v[pl.ds(g * 16, 16)]
        pads_r = jnp.max(jnp.where(lane == l, pvec, 0))
        return _L - pads_r

    def gather_plan(n):
        nfull = jnp.int32(0)
        for k in range(1, _NCH + 1):
            nfull = nfull + (n >= k * _CH).astype(jnp.int32)
        m = n - nfull * _CH
        tail8 = lax.shift_right_logical(m, 3)
        rem = m & 7
        return nfull, tail8, rem

    def each_gather(r, buf, n, go):
        nfull, tail8, rem = gather_plan(n)
        for c in range(_NCH):
            @pl.when(c < nfull)
            def _():
                go(table_hbm.at[ids_v.at[pl.ds(r * _L + c * _CH, _CH)]],
                   buf.at[pl.ds(c * _CH, _CH)])
        for t in range(1, _CH // 8):
            @pl.when(tail8 == t)
            def _():
                go(table_hbm.at[ids_v.at[pl.ds(r * _L + nfull * _CH, 8 * t)]],
                   buf.at[pl.ds(nfull * _CH, 8 * t)])
        off = nfull * _CH + tail8 * 8
        for q in range(1, 8):
            @pl.when(rem == q)
            def _():
                go(table_hbm.at[ids_v.at[pl.ds(r * _L + off, q)]],
                   buf.at[pl.ds(off, q)])

    def fire_row(r, buf, sem, n):
        each_gather(r, buf, n,
                    lambda src, dst: pltpu.async_copy(src, dst, sem))

    def compute_row(r, buf, sem, p, semw, pv, n):
        each_gather(r, buf, n,
                    lambda src, dst: pltpu.make_async_copy(src, dst,
                                                           sem).wait())

        pads_r = _L - n
        acc = ((zero,) * _NV,
               tuple(jnp.where(pads_r > 0, pv[v], neg_inf)
                     for v in range(_NV)))

        def body(j, acc2):
            sums, maxs = acc2
            ra = tuple(buf[j, pl.ds(16 * v, 16)] for v in range(_NV))
            rb = tuple(buf[j + 1, pl.ds(16 * v, 16)] for v in range(_NV))
            return (tuple(sums[v] + (ra[v] + rb[v]) for v in range(_NV)),
                    tuple(jnp.maximum(maxs[v], jnp.maximum(ra[v], rb[v]))
                          for v in range(_NV)))

        odd = (n & 1) == 1
        n_even = n - (n & 1)
        sums, maxs = plsc.parallel_loop(0, n_even, 2, unroll=2,
                                        carry=acc)(body)
        rt = tuple(buf[n - 1, pl.ds(16 * v, 16)] for v in range(_NV))
        sums = tuple(sums[v] + jnp.where(odd, rt[v], zero)
                     for v in range(_NV))
        maxs = tuple(jnp.maximum(maxs[v], jnp.where(odd, rt[v], neg_inf))
                     for v in range(_NV))
        n_vec = jnp.full((16,), 1.0, jnp.float32) * n.astype(jnp.float32)
        inv = jnp.full((16,), 1.0, jnp.float32) / n_vec
        @pl.when(r >= 2)
        def _():
            pltpu.make_async_copy(outrow_v.at[p], out_hbm.at[base + r - 2],
                                  semw).wait()
        for v in range(_NV):
            outrow_v[p, pl.ds(16 * v, 16)] = maxs[v]
            outrow_v[p, pl.ds(_DIM + 16 * v, 16)] = sums[v] * inv
        pltpu.async_copy(outrow_v.at[p], out_hbm.at[base + r], semw)

    bufs = (buf0_v, buf1_v, buf2_v)
    sems = (sem0, sem1, sem2)

    def fire_guarded(r, buf, sem):
        @pl.when(r < _ROWS_PER_W)
        def _():
            fire_row(r, buf, sem, seq_len(jnp.minimum(r, _ROWS_PER_W - 1)))

    fire_row(jnp.int32(0), buf0_v, sem0, seq_len(jnp.int32(0)))
    fire_row(jnp.int32(1), buf1_v, sem1, seq_len(jnp.int32(1)))

    def triple_body(tp, pv_c):
        r0 = tp * 3
        for k in range(3):
            fire_guarded(r0 + k + 2, bufs[(k + 2) % 3], sems[(k + 2) % 3])
            compute_row(r0 + k, bufs[k], sems[k], k & 1, (semw0, semw1)[k & 1],
                        pv_c, seq_len(r0 + k))
        return pv_c

    lax.fori_loop(0, _ROWS_PER_W // 3, triple_body, pv)
    r126 = jnp.int32(_ROWS_PER_W - 2)
    r127 = jnp.int32(_ROWS_PER_W - 1)
    compute_row(r126, buf0_v, sem0, 0, semw0, pv, seq_len(r126))
    compute_row(r127, buf1_v, sem1, 1, semw1, pv, seq_len(r127))
    pltpu.make_async_copy(outrow_v.at[0],
                          out_hbm.at[base + _ROWS_PER_W - 2], semw0).wait()
    pltpu.make_async_copy(outrow_v.at[1],
                          out_hbm.at[base + _ROWS_PER_W - 1], semw1).wait()


@jax.jit
def _pooler(ids2, pads, emb_table):
    mesh = plsc.VectorSubcoreMesh(core_axis_name="c", subcore_axis_name="s")
    run = pl.kernel(
        _pooler_body,
        out_type=jax.ShapeDtypeStruct((_B, 2 * _DIM), jnp.float32),
        mesh=mesh,
        scratch_types=[
            pltpu.VMEM((_ROWS_PER_W * _L,), jnp.int32),
            pltpu.VMEM((_ROWS_PER_W,), jnp.int32),
            pltpu.VMEM((_L, _DIM), jnp.float32),
            pltpu.VMEM((_L, _DIM), jnp.float32),
            pltpu.VMEM((_L, _DIM), jnp.float32),
            pltpu.VMEM((2, 2 * _DIM), jnp.float32),
            pltpu.SemaphoreType.DMA,
            pltpu.SemaphoreType.DMA,
            pltpu.SemaphoreType.DMA,
            pltpu.SemaphoreType.DMA,
            pltpu.SemaphoreType.DMA,
        ],
        compiler_params=pltpu.CompilerParams(needs_layout_passes=False),
    )
    return run(ids2, pads, emb_table)


def kernel(ids, pads, emb_table):
    ids2 = ids.reshape(_B * _L)
    return _pooler(ids2, pads, emb_table)


# speedup vs baseline: 1.1693x; 1.0015x over previous
import jax
import jax.numpy as jnp
from jax import lax
from jax.experimental import pallas as pl
from jax.experimental.pallas import tpu as pltpu
from jax.experimental.pallas import tpu_sc as plsc

_VOCAB = 100000
_N_EMB = _VOCAB + 2
_DIM = 128
_B = 4096
_L = 200
_PADDING_ID = _N_EMB - 1

_CH = 48
_NCH = _L // _CH
_NV = _DIM // 16

_NWORKERS = 32
_ROWS_PER_W = _B // _NWORKERS


def _pooler_body(ids_hbm, pads_hbm, table_hbm, out_hbm,
                 ids_v, pads_v, buf0_v, buf1_v, buf2_v, outrow_v,
                 sem0, sem1, sem2, semw0, semw1):
    nc = 2
    wid = lax.axis_index("s") * nc + lax.axis_index("c")
    base = wid * _ROWS_PER_W

    pltpu.sync_copy(ids_hbm.at[pl.ds(base * _L, _ROWS_PER_W * _L)], ids_v)
    pltpu.sync_copy(pads_hbm.at[pl.ds(base, _ROWS_PER_W)], pads_v)
    pltpu.sync_copy(table_hbm.at[pl.ds(_PADDING_ID, 1)],
                    outrow_v.at[pl.ds(0, 1), pl.ds(0, _DIM)])
    pv = tuple(outrow_v[0, pl.ds(16 * v, 16)] for v in range(_NV))

    neg_inf = jnp.full((16,), -jnp.inf, jnp.float32)
    zero = jnp.zeros((16,), jnp.float32)
    lane = lax.iota(jnp.int32, 16)

    def seq_len(r):
        g = lax.shift_right_logical(r, 4)
        l = r - g * 16
        pvec = pads_v[pl.ds(g * 16, 16)]
        pads_r = jnp.max(jnp.where(lane == l, pvec, 0))
        return _L - pads_r

    def gather_plan(n):
        nfull = jnp.int32(0)
        for k in range(1, _NCH + 1):
            nfull = nfull + (n >= k * _CH).astype(jnp.int32)
        m = n - nfull * _CH
        tail8 = lax.shift_right_logical(m, 3)
        rem = m & 7
        return nfull, tail8, rem

    def each_gather(r, buf, n, go):
        nfull, tail8, rem = gather_plan(n)
        for c in range(_NCH):
            @pl.when(c < nfull)
            def _():
                go(table_hbm.at[ids_v.at[pl.ds(r * _L + c * _CH, _CH)]],
                   buf.at[pl.ds(c * _CH, _CH)])
        for t in range(1, _CH // 8):
            @pl.when(tail8 == t)
            def _():
                go(table_hbm.at[ids_v.at[pl.ds(r * _L + nfull * _CH, 8 * t)]],
                   buf.at[pl.ds(nfull * _CH, 8 * t)])
        off = nfull * _CH + tail8 * 8
        for q in range(1, 8):
            @pl.when(rem == q)
            def _():
                go(table_hbm.at[ids_v.at[pl.ds(r * _L + off, q)]],
                   buf.at[pl.ds(off, q)])

    def fire_row(r, buf, sem, n):
        each_gather(r, buf, n,
                    lambda src, dst: pltpu.async_copy(src, dst, sem))

    def compute_row(r, buf, sem, p, semw, pv, n):
        each_gather(r, buf, n,
                    lambda src, dst: pltpu.make_async_copy(src, dst,
                                                           sem).wait())

        pads_r = _L - n
        acc = ((zero,) * _NV,
               tuple(jnp.where(pads_r > 0, pv[v], neg_inf)
                     for v in range(_NV)))

        def body(j, acc2):
            sums, maxs = acc2
            ra = tuple(buf[j, pl.ds(16 * v, 16)] for v in range(_NV))
            rb = tuple(buf[j + 1, pl.ds(16 * v, 16)] for v in range(_NV))
            return (tuple(sums[v] + (ra[v] + rb[v]) for v in range(_NV)),
                    tuple(jnp.maximum(maxs[v], jnp.maximum(ra[v], rb[v]))
                          for v in range(_NV)))

        odd = (n & 1) == 1
        n_even = n - (n & 1)
        sums, maxs = plsc.parallel_loop(0, n_even, 2, unroll=2,
                                        carry=acc)(body)
        rt = tuple(buf[n - 1, pl.ds(16 * v, 16)] for v in range(_NV))
        sums = tuple(sums[v] + jnp.where(odd, rt[v], zero)
                     for v in range(_NV))
        maxs = tuple(jnp.maximum(maxs[v], jnp.where(odd, rt[v], neg_inf))
                     for v in range(_NV))
        n_vec = jnp.full((16,), 1.0, jnp.float32) * n.astype(jnp.float32)
        inv = jnp.full((16,), 1.0, jnp.float32) / n_vec
        @pl.when(r >= 2)
        def _():
            pltpu.make_async_copy(outrow_v.at[p], out_hbm.at[base + r - 2],
                                  semw).wait()
        for v in range(_NV):
            outrow_v[p, pl.ds(16 * v, 16)] = maxs[v]
            outrow_v[p, pl.ds(_DIM + 16 * v, 16)] = sums[v] * inv
        pltpu.async_copy(outrow_v.at[p], out_hbm.at[base + r], semw)

    bufs = (buf0_v, buf1_v, buf2_v)
    sems = (sem0, sem1, sem2)

    def fire_guarded(r, buf, sem):
        @pl.when(r < _ROWS_PER_W)
        def _():
            fire_row(r, buf, sem, seq_len(jnp.minimum(r, _ROWS_PER_W - 1)))

    fire_row(jnp.int32(0), buf0_v, sem0, seq_len(jnp.int32(0)))
    fire_row(jnp.int32(1), buf1_v, sem1, seq_len(jnp.int32(1)))

    def triple_body(tp, pv_c):
        r0 = tp * 3
        for k in range(3):
            fire_guarded(r0 + k + 2, bufs[(k + 2) % 3], sems[(k + 2) % 3])
            compute_row(r0 + k, bufs[k], sems[k], k & 1, (semw0, semw1)[k & 1],
                        pv_c, seq_len(r0 + k))
        return pv_c

    lax.fori_loop(0, _ROWS_PER_W // 3, triple_body, pv)
    r126 = jnp.int32(_ROWS_PER_W - 2)
    r127 = jnp.int32(_ROWS_PER_W - 1)
    compute_row(r126, buf0_v, sem0, 0, semw0, pv, seq_len(r126))
    compute_row(r127, buf1_v, sem1, 1, semw1, pv, seq_len(r127))
    pltpu.make_async_copy(outrow_v.at[0],
                          out_hbm.at[base + _ROWS_PER_W - 2], semw0).wait()
    pltpu.make_async_copy(outrow_v.at[1],
                          out_hbm.at[base + _ROWS_PER_W - 1], semw1).wait()


@jax.jit
def _pooler(ids2, pads, emb_table):
    mesh = plsc.VectorSubcoreMesh(core_axis_name="c", subcore_axis_name="s")
    run = pl.kernel(
        _pooler_body,
        out_type=jax.ShapeDtypeStruct((_B, 2 * _DIM), jnp.float32),
        mesh=mesh,
        scratch_types=[
            pltpu.VMEM((_ROWS_PER_W * _L,), jnp.int32),
            pltpu.VMEM((_ROWS_PER_W,), jnp.int32),
            pltpu.VMEM((_L, _DIM), jnp.float32),
            pltpu.VMEM((_L, _DIM), jnp.float32),
            pltpu.VMEM((_L, _DIM), jnp.float32),
            pltpu.VMEM((2, 2 * _DIM), jnp.float32),
            pltpu.SemaphoreType.DMA,
            pltpu.SemaphoreType.DMA,
            pltpu.SemaphoreType.DMA,
            pltpu.SemaphoreType.DMA,
            pltpu.SemaphoreType.DMA,
        ],
        compiler_params=pltpu.CompilerParams(needs_layout_passes=False),
    )
    return run(ids2, pads, emb_table)


def kernel(ids, pads, emb_table):
    ids2 = ids.reshape(_B * _L)
    return _pooler(ids2, pads, emb_table)
